# Initial kernel scaffold; baseline (speedup 1.0000x reference)
import jax
import jax.numpy as jnp
from jax.experimental import pallas as pl


def kernel(feat, segment_ids, Wa, ba, Wl, bl):
    raise NotImplementedError("write your pallas kernel here")


# speedup vs baseline: 2.4631x; 2.4631x over previous
import functools

import jax
import jax.numpy as jnp
from jax import lax
from jax.experimental import pallas as pl
from jax.experimental.pallas import tpu as pltpu
from jax.experimental.pallas import tpu_sc as plsc

N = 50000
F = 256
B = 128
OUT = 256
NW = 32
R = 64
NCH = 25
NPAIR = (NCH - 1) // 2
SEGBUF = 1576
L = 16
NG = N // 8
GBASE = NG // NW
GREM = NG % NW


def _sc_body(feat_h, seg_h, wa_h, psum_h, pmax_h,
             buf0, buf1, segbuf, wabuf, accsum, accmax, sem0, sem1):
    cid = lax.axis_index("c")
    sid = lax.axis_index("s")
    wid = sid * 2 + cid
    s = 8 * (wid * GBASE + jnp.minimum(wid, GREM))
    cnt = 8 * (GBASE + (wid < GREM).astype(jnp.int32))
    e = s + cnt
    a = jnp.minimum((s // 8) * 8, N - SEGBUF)

    pltpu.sync_copy(wa_h, wabuf)
    pltpu.sync_copy(seg_h.at[pl.ds(a, SEGBUF)], segbuf)

    zero16 = jnp.zeros((L,), jnp.float32)
    ninf16 = jnp.full((L,), -jnp.inf, jnp.float32)
    iota16 = lax.iota(jnp.int32, L)
    bav = wabuf[pl.ds(F, L)]

    def init_body(i, _):
        accsum[pl.ds(i * L, L)] = zero16
        accmax[pl.ds(i * L, L)] = ninf16
        return 0
    lax.fori_loop(0, (B * F) // L, init_body, 0)

    def chunk_base(i):
        return jnp.minimum(s + i * R, e - R)

    def start_copy(i, buf, sem):
        pltpu.async_copy(feat_h.at[pl.ds(chunk_base(i), R)], buf, sem)

    def wait_copy(buf, sem):
        pltpu.make_async_copy(feat_h.at[pl.ds(0, R)], buf, sem).wait()

    def process_chunk(i, buf):
        cc = chunk_base(i)
        lo = (s + i * R) - cc

        def row_body(k, _):
            rg = cc + k
            segi = plsc.load_gather(segbuf, [jnp.full((L,), rg - a, jnp.int32)])
            dot = zero16
            for j in range(F // L):
                v = buf[k, pl.ds(j * L, L)]
                dot = dot + v * wabuf[pl.ds(j * L, L)]
            tv = jnp.full((L,), jnp.sum(dot)) + bav
            wv = 1.0 / (1.0 + jnp.exp(-tv))
            base = segi * F + iota16
            for j in range(F // L):
                v = buf[k, pl.ds(j * L, L)]
                idx = base + (j * L)
                plsc.addupdate_scatter(accsum, [idx], v * wv)
                m = plsc.load_gather(accmax, [idx])
                plsc.store_scatter(accmax, [idx], jnp.maximum(m, v))
            return 0
        lax.fori_loop(lo, R, row_body, 0)

    start_copy(0, buf0, sem0)

    def g_body(g, _):
        start_copy(2 * g + 1, buf1, sem1)
        wait_copy(buf0, sem0)
        process_chunk(2 * g, buf0)
        start_copy(2 * g + 2, buf0, sem0)
        wait_copy(buf1, sem1)
        process_chunk(2 * g + 1, buf1)
        return 0
    lax.fori_loop(0, NPAIR, g_body, 0)
    wait_copy(buf0, sem0)
    process_chunk(NCH - 1, buf0)

    pltpu.sync_copy(accsum, psum_h.at[pl.ds(wid * (B * F), B * F)])
    pltpu.sync_copy(accmax, pmax_h.at[pl.ds(wid * (B * F), B * F)])


_sc_pool = functools.partial(
    pl.kernel,
    mesh=plsc.VectorSubcoreMesh(core_axis_name="c", subcore_axis_name="s"),
    out_type=(
        jax.ShapeDtypeStruct((NW * B * F,), jnp.float32),
        jax.ShapeDtypeStruct((NW * B * F,), jnp.float32),
    ),
    scratch_types=[
        pltpu.VMEM((R, F), jnp.float32),
        pltpu.VMEM((R, F), jnp.float32),
        pltpu.VMEM((SEGBUF,), jnp.int32),
        pltpu.VMEM((F + L,), jnp.float32),
        pltpu.VMEM((B * F,), jnp.float32),
        pltpu.VMEM((B * F,), jnp.float32),
        pltpu.SemaphoreType.DMA,
        pltpu.SemaphoreType.DMA,
    ],
    compiler_params=pltpu.CompilerParams(needs_layout_passes=False),
)(_sc_body)


def _tc_body(psum_ref, pmax_ref, wl_ref, bl_ref, out_ref):
    hs = jnp.sum(psum_ref[...], axis=0)
    hm = jnp.max(pmax_ref[...], axis=0)
    acc = jnp.dot(hs, wl_ref[0:F, :], preferred_element_type=jnp.float32)
    acc = acc + jnp.dot(hm, wl_ref[F:2 * F, :], preferred_element_type=jnp.float32)
    out_ref[...] = acc + bl_ref[...]


_tc_finish = pl.pallas_call(
    _tc_body,
    out_shape=jax.ShapeDtypeStruct((B, OUT), jnp.float32),
)


def kernel(feat, segment_ids, Wa, ba, Wl, bl):
    seg32 = segment_ids.astype(jnp.int32)
    wa_ext = jnp.concatenate(
        [Wa.reshape(F), jnp.full((L,), ba[0])]).astype(jnp.float32)
    psum, pmax = _sc_pool(feat, seg32, wa_ext)
    return _tc_finish(psum.reshape(NW, B, F), pmax.reshape(NW, B, F),
                      Wl, bl.reshape(1, OUT))


# speedup vs baseline: 4.7198x; 1.9162x over previous
import functools

import jax
import jax.numpy as jnp
from jax import lax
from jax.experimental import pallas as pl
from jax.experimental.pallas import tpu as pltpu
from jax.experimental.pallas import tpu_sc as plsc

N = 50000
F = 256
B = 128
OUT = 256
NW = 32
R = 64
NCH = 25
NPAIR = (NCH - 1) // 2
L = 16
NJ = F // L
SEGDMA = 1624
SEGBUF = SEGDMA + L
NG = N // 8
GBASE = NG // NW
GREM = NG % NW


def _sc_body(feat_h, seg_h, wa_h, psum_h, pmax_h,
             buf0, buf1, segbuf, wabuf, accsum, accmax, sem0, sem1):
    cid = lax.axis_index("c")
    sid = lax.axis_index("s")
    wid = sid * 2 + cid
    s = 8 * (wid * GBASE + jnp.minimum(wid, GREM))
    cnt = 8 * (GBASE + (wid < GREM).astype(jnp.int32))
    e = s + cnt
    a = jnp.minimum((s // 8) * 8, N - SEGDMA)

    pltpu.sync_copy(wa_h, wabuf)
    pltpu.sync_copy(seg_h.at[pl.ds(a, SEGDMA)], segbuf.at[pl.ds(0, SEGDMA)])

    zero16 = jnp.zeros((L,), jnp.float32)
    ninf16 = jnp.full((L,), -jnp.inf, jnp.float32)
    bav = wabuf[pl.ds(F, L)]

    def init_body(i, _):
        accsum[pl.ds(i * L, L)] = zero16
        accmax[pl.ds(i * L, L)] = ninf16
        return 0
    lax.fori_loop(0, (B * F) // L, init_body, 0)

    def chunk_base(i):
        return jnp.minimum(s + i * R, e - R)

    def start_copy(i, buf, sem):
        pltpu.async_copy(feat_h.at[pl.ds(chunk_base(i), R)], buf, sem)

    def wait_copy(buf, sem):
        pltpu.make_async_copy(feat_h.at[pl.ds(0, R)], buf, sem).wait()

    def seg_of(rg):
        return segbuf[pl.ds(rg - a, L)][0]

    def flush(carry):
        ps = carry[0]
        base = ps * F
        for j in range(NJ):
            accsum[pl.ds(base + j * L, L)] = carry[1 + j]
            accmax[pl.ds(base + j * L, L)] = carry[1 + NJ + j]
        return (jnp.int32(0),) + (zero16,) * NJ + (ninf16,) * NJ

    def process_chunk(i, buf, carry):
        cc = chunk_base(i)
        lo = (s + i * R) - cc

        def row_body(k, carry):
            ps = carry[0]
            sr = seg_of(cc + k)

            def do_flush(c):
                out = flush(c)
                return (sr,) + out[1:]

            def keep(c):
                return (sr,) + c[1:]

            carry = lax.cond(sr != ps, do_flush, keep, carry)
            sums = list(carry[1:1 + NJ])
            maxs = list(carry[1 + NJ:1 + 2 * NJ])
            dot = zero16
            for j in range(NJ):
                v = buf[k, pl.ds(j * L, L)]
                dot = dot + v * wabuf[pl.ds(j * L, L)]
                maxs[j] = jnp.maximum(maxs[j], v)
            tv = jnp.full((L,), jnp.sum(dot)) + bav
            wv = 1.0 / (1.0 + jnp.exp(-tv))
            for j in range(NJ):
                v = buf[k, pl.ds(j * L, L)]
                sums[j] = sums[j] + v * wv
            return (sr,) + tuple(sums) + tuple(maxs)

        return lax.fori_loop(lo, R, row_body, carry)

    carry0 = (seg_of(s),) + (zero16,) * NJ + (ninf16,) * NJ

    start_copy(0, buf0, sem0)

    def g_body(g, carry):
        start_copy(2 * g + 1, buf1, sem1)
        wait_copy(buf0, sem0)
        carry = process_chunk(2 * g, buf0, carry)
        start_copy(2 * g + 2, buf0, sem0)
        wait_copy(buf1, sem1)
        carry = process_chunk(2 * g + 1, buf1, carry)
        return carry

    carry = lax.fori_loop(0, NPAIR, g_body, carry0)
    wait_copy(buf0, sem0)
    carry = process_chunk(NCH - 1, buf0, carry)
    flush(carry)

    pltpu.sync_copy(accsum, psum_h.at[pl.ds(wid * (B * F), B * F)])
    pltpu.sync_copy(accmax, pmax_h.at[pl.ds(wid * (B * F), B * F)])


_sc_pool = functools.partial(
    pl.kernel,
    mesh=plsc.VectorSubcoreMesh(core_axis_name="c", subcore_axis_name="s"),
    out_type=(
        jax.ShapeDtypeStruct((NW * B * F,), jnp.float32),
        jax.ShapeDtypeStruct((NW * B * F,), jnp.float32),
    ),
    scratch_types=[
        pltpu.VMEM((R, F), jnp.float32),
        pltpu.VMEM((R, F), jnp.float32),
        pltpu.VMEM((SEGBUF,), jnp.int32),
        pltpu.VMEM((F + L,), jnp.float32),
        pltpu.VMEM((B * F,), jnp.float32),
        pltpu.VMEM((B * F,), jnp.float32),
        pltpu.SemaphoreType.DMA,
        pltpu.SemaphoreType.DMA,
    ],
    compiler_params=pltpu.CompilerParams(needs_layout_passes=False),
)(_sc_body)


def _tc_body(psum_ref, pmax_ref, wl_ref, bl_ref, out_ref):
    hs = jnp.sum(psum_ref[...], axis=0)
    hm = jnp.max(pmax_ref[...], axis=0)
    acc = jnp.dot(hs, wl_ref[0:F, :], preferred_element_type=jnp.float32)
    acc = acc + jnp.dot(hm, wl_ref[F:2 * F, :], preferred_element_type=jnp.float32)
    out_ref[...] = acc + bl_ref[...]


_tc_finish = pl.pallas_call(
    _tc_body,
    out_shape=jax.ShapeDtypeStruct((B, OUT), jnp.float32),
)


def kernel(feat, segment_ids, Wa, ba, Wl, bl):
    seg32 = segment_ids.astype(jnp.int32)
    wa_ext = jnp.concatenate(
        [Wa.reshape(F), jnp.full((L,), ba[0])]).astype(jnp.float32)
    psum, pmax = _sc_pool(feat, seg32, wa_ext)
    return _tc_finish(psum.reshape(NW, B, F), pmax.reshape(NW, B, F),
                      Wl, bl.reshape(1, OUT))
